# baseline (speedup 1.0000x reference)
import functools

import jax
import jax.numpy as jnp
from jax import lax
from jax.experimental import pallas as pl
from jax.experimental.pallas import tpu as pltpu

LAMBDA_PARAM = 25.0
MU_PARAM = 25.0
NU_PARAM = 1.0
ALPHA = 0.75
EPS = 1e-4
K = 20
L = 576
C = 768
D = 2048
BIG = 3.0e9

_NT = (((1,), (1,)), ((), ()))
_TN = (((0,), (0,)), ((), ()))


def _fiota(shape, dim):
    return lax.broadcasted_iota(jnp.int32, shape, dim).astype(jnp.float32)


def _dot(a, b, dims):
    return lax.dot_general(a, b, dims, precision=lax.Precision.HIGHEST)


def _cdist_sq(za, zb):
    a2 = jnp.sum(za * za, axis=1, keepdims=True)
    ones = jnp.ones((1, za.shape[1]), jnp.float32)
    b2t = _dot(ones, zb * zb, _NT)
    d2 = a2 + b2t - 2.0 * _dot(za, zb, _NT)
    return jnp.sqrt(jnp.maximum(d2, 1e-12))


def _select_rows(dist, zin, zcand):
    col = _fiota((L, L), 1)
    nnv = jnp.min(dist, axis=1, keepdims=True)
    nni = jnp.min(jnp.where(dist == nnv, col, BIG), axis=1, keepdims=True)
    row = _fiota((L, 1), 0)
    vals = nnv
    cols = []
    for _ in range(K):
        m = jnp.min(vals, axis=0, keepdims=True)
        idx = jnp.min(jnp.where(vals == m, row, BIG), axis=0, keepdims=True)
        hit = row == idx
        cols.append(hit.astype(jnp.float32))
        vals = jnp.where(hit, BIG, vals)
    s = jnp.concatenate(cols, axis=1)
    xin = _dot(s, zin, _TN)
    cand_f = _dot(s, nni, _TN)
    t = (_fiota((K, L), 1) == cand_f).astype(jnp.float32)
    xcand = _dot(t, zcand, (((1,), (0,)), ((), ())))
    return xin, xcand


def _select_cols(dist, zin, zcand):
    rowi = _fiota((L, L), 0)
    nnv = jnp.min(dist, axis=0, keepdims=True)
    nni = jnp.min(jnp.where(dist == nnv, rowi, BIG), axis=0, keepdims=True)
    col = _fiota((1, L), 1)
    vals = nnv
    rows = []
    for _ in range(K):
        m = jnp.min(vals, axis=1, keepdims=True)
        idx = jnp.min(jnp.where(vals == m, col, BIG), axis=1, keepdims=True)
        hit = col == idx
        rows.append(hit.astype(jnp.float32))
        vals = jnp.where(hit, BIG, vals)
    s = jnp.concatenate(rows, axis=0)
    xin = _dot(s, zin, (((1,), (0,)), ((), ())))
    cand_f = _dot(s, nni, _NT)
    t = (_fiota((K, L), 1) == cand_f).astype(jnp.float32)
    xcand = _dot(t, zcand, (((1,), (0,)), ((), ())))
    return xin, xcand


def _match_kernel(za_ref, zb_ref, ga_ref, gb_ref, fa_ref, na_ref):
    za = za_ref[0]
    zb = zb_ref[0]
    ga = ga_ref[0]
    gb = gb_ref[0]

    dist_f = _cdist_sq(za, zb)
    dist_g = _cdist_sq(ga, gb)

    for p, (dist, fwd) in enumerate((
            (dist_f, True), (dist_f, False), (dist_g, True), (dist_g, False))):
        if fwd:
            xin, xcand = _select_rows(dist, za, zb)
        else:
            xin, xcand = _select_cols(dist, zb, za)
        fa_ref[p, 0] = xin
        na_ref[p, 0] = xcand


def _var_loss(xc, n):
    var = jnp.sum(xc * xc, axis=0, keepdims=True) / (n - 1.0)
    std = jnp.sqrt(var + EPS)
    return jnp.mean(jnp.maximum(1.0 - std, 0.0))


def _cov_loss_direct(xc, n, d):
    m = _dot(xc, xc, _TN)
    msq = m * m
    diag = lax.broadcasted_iota(jnp.int32, msq.shape, 0) == \
        lax.broadcasted_iota(jnp.int32, msq.shape, 1)
    off = jnp.sum(jnp.where(diag, 0.0, msq))
    return off / ((n - 1.0) ** 2 * d)


def _cov_loss_gram(xc, n, d):
    g = _dot(xc, xc, _NT)
    s = jnp.sum(xc * xc, axis=0, keepdims=True)
    off = jnp.sum(g * g) - jnp.sum(s * s)
    return off / ((n - 1.0) ** 2 * d)


def _vicreg_pair(xa, xb, n, d, gram):
    inv = jnp.sum((xa - xb) ** 2) / (n * d)
    xca = xa - jnp.mean(xa, axis=0, keepdims=True)
    xcb = xb - jnp.mean(xb, axis=0, keepdims=True)
    var = 0.5 * (_var_loss(xca, n) + _var_loss(xcb, n))
    covf = _cov_loss_gram if gram else _cov_loss_direct
    cov = covf(xca, n, d) + covf(xcb, n, d)
    return LAMBDA_PARAM * inv + MU_PARAM * var + NU_PARAM * cov


def _vicreg_kernel(fa_ref, na_ref, g0_ref, g1_ref, out_ref, acc_ref):
    p = pl.program_id(0)

    @pl.when(p == 0)
    def _():
        acc_ref[0] = ALPHA * _vicreg_pair(g0_ref[...], g1_ref[...], 64.0, float(D), True)

    n = float(64 * K)
    acc_ref[0] += (1.0 - ALPHA) * 0.5 * _vicreg_pair(
        fa_ref[0], na_ref[0], n, float(C), False)

    @pl.when(p == 3)
    def _():
        out_ref[0] = acc_ref[0]


@jax.jit
def _vicregl(z_global0, z_global1, z_local0, z_local1, grid0, grid1):
    B = z_local0.shape[0]
    za = z_local0.reshape(B, L, C)
    zb = z_local1.reshape(B, L, C)
    ga = grid0.reshape(B, L, 2)
    gb = grid1.reshape(B, L, 2)

    fa, na = pl.pallas_call(
        _match_kernel,
        grid=(B,),
        in_specs=[
            pl.BlockSpec((1, L, C), lambda b: (b, 0, 0)),
            pl.BlockSpec((1, L, C), lambda b: (b, 0, 0)),
            pl.BlockSpec((1, L, 2), lambda b: (b, 0, 0)),
            pl.BlockSpec((1, L, 2), lambda b: (b, 0, 0)),
        ],
        out_specs=[
            pl.BlockSpec((4, 1, K, C), lambda b: (0, b, 0, 0)),
            pl.BlockSpec((4, 1, K, C), lambda b: (0, b, 0, 0)),
        ],
        out_shape=[
            jax.ShapeDtypeStruct((4, B, K, C), jnp.float32),
            jax.ShapeDtypeStruct((4, B, K, C), jnp.float32),
        ],
    )(za, zb, ga, gb)

    fa = fa.reshape(4, B * K, C)
    na = na.reshape(4, B * K, C)

    out = pl.pallas_call(
        _vicreg_kernel,
        grid=(4,),
        in_specs=[
            pl.BlockSpec((1, B * K, C), lambda p: (p, 0, 0)),
            pl.BlockSpec((1, B * K, C), lambda p: (p, 0, 0)),
            pl.BlockSpec((B, D), lambda p: (0, 0)),
            pl.BlockSpec((B, D), lambda p: (0, 0)),
        ],
        out_specs=pl.BlockSpec(memory_space=pltpu.SMEM),
        out_shape=jax.ShapeDtypeStruct((1,), jnp.float32),
        scratch_shapes=[pltpu.SMEM((1,), jnp.float32)],
    )(fa, na, z_global0, z_global1)

    return out[0]


def kernel(z_global0, z_global1, z_local0, z_local1, grid0, grid1):
    return _vicregl(z_global0, z_global1, z_local0, z_local1, grid0, grid1)


# speedup vs baseline: 2.6256x; 2.6256x over previous
import functools

import jax
import jax.numpy as jnp
from jax import lax
from jax.experimental import pallas as pl
from jax.experimental.pallas import tpu as pltpu

LAMBDA_PARAM = 25.0
MU_PARAM = 25.0
NU_PARAM = 1.0
ALPHA = 0.75
EPS = 1e-4
K = 20
L = 576
C = 768
D = 2048
BIG = 3.0e9

_NT = (((1,), (1,)), ((), ()))
_TN = (((0,), (0,)), ((), ()))


def _fiota(shape, dim):
    return lax.broadcasted_iota(jnp.int32, shape, dim).astype(jnp.float32)


def _dot(a, b, dims):
    return lax.dot_general(a, b, dims, precision=lax.Precision.DEFAULT)


def _cdist_sq(za, zb):
    a2 = jnp.sum(za * za, axis=1, keepdims=True)
    ones = jnp.ones((1, za.shape[1]), jnp.float32)
    b2t = _dot(ones, zb * zb, _NT)
    d2 = a2 + b2t - 2.0 * _dot(za, zb, _NT)
    return jnp.maximum(d2, 1e-12)


def _topk_onehots(nnv4):
    col = _fiota((4, L), 1)
    vals = nnv4
    hits = []
    for _ in range(K):
        m = jnp.min(vals, axis=1, keepdims=True)
        idx = jnp.min(jnp.where(vals == m, col, BIG), axis=1, keepdims=True)
        hit = col == idx
        hits.append(hit.astype(jnp.float32))
        vals = jnp.where(hit, BIG, vals)
    return hits


def _select_pairs(s, dist, nnv, zin, zcand, ddim):
    dsel = _dot(s, dist, (((1,), (ddim,)), ((), ())))
    nnv_sel = _dot(s, nnv, _NT)
    kcol = _fiota((K, L), 1)
    cand_f = jnp.min(jnp.where(dsel == nnv_sel, kcol, BIG), axis=1, keepdims=True)
    t = (kcol == cand_f).astype(jnp.float32)
    xin = _dot(s, zin, (((1,), (0,)), ((), ())))
    xcand = _dot(t, zcand, (((1,), (0,)), ((), ())))
    return xin, xcand


def _match_kernel(za_ref, zb_ref, ga_ref, gb_ref, fa_ref, na_ref):
    za = za_ref[0]
    zb = zb_ref[0]
    ga = ga_ref[0]
    gb = gb_ref[0]

    dist_f = _cdist_sq(za, zb)
    dist_g = _cdist_sq(ga, gb)

    nnv4 = jnp.concatenate([
        lax.transpose(jnp.min(dist_f, axis=1, keepdims=True), (1, 0)),
        jnp.min(dist_f, axis=0, keepdims=True),
        lax.transpose(jnp.min(dist_g, axis=1, keepdims=True), (1, 0)),
        jnp.min(dist_g, axis=0, keepdims=True),
    ], axis=0)
    hits = _topk_onehots(nnv4)

    for p, (dist, ddim, zin, zcand) in enumerate((
            (dist_f, 0, za, zb), (dist_f, 1, zb, za),
            (dist_g, 0, za, zb), (dist_g, 1, zb, za))):
        s = jnp.concatenate([h[p:p + 1] for h in hits], axis=0)
        nnv = nnv4[p:p + 1]
        xin, xcand = _select_pairs(s, dist, nnv, zin, zcand, ddim)
        fa_ref[p, 0] = xin
        na_ref[p, 0] = xcand


def _var_loss(xc, n):
    var = jnp.sum(xc * xc, axis=0, keepdims=True) / (n - 1.0)
    std = jnp.sqrt(var + EPS)
    return jnp.mean(jnp.maximum(1.0 - std, 0.0))


def _cov_loss_direct(xc, n, d):
    m = _dot(xc, xc, _TN)
    msq = m * m
    diag = lax.broadcasted_iota(jnp.int32, msq.shape, 0) == \
        lax.broadcasted_iota(jnp.int32, msq.shape, 1)
    off = jnp.sum(jnp.where(diag, 0.0, msq))
    return off / ((n - 1.0) ** 2 * d)


def _cov_loss_gram(xc, n, d):
    g = _dot(xc, xc, _NT)
    s = jnp.sum(xc * xc, axis=0, keepdims=True)
    off = jnp.sum(g * g) - jnp.sum(s * s)
    return off / ((n - 1.0) ** 2 * d)


def _vicreg_pair(xa, xb, n, d, gram):
    inv = jnp.sum((xa - xb) ** 2) / (n * d)
    xca = xa - jnp.mean(xa, axis=0, keepdims=True)
    xcb = xb - jnp.mean(xb, axis=0, keepdims=True)
    var = 0.5 * (_var_loss(xca, n) + _var_loss(xcb, n))
    covf = _cov_loss_gram if gram else _cov_loss_direct
    cov = covf(xca, n, d) + covf(xcb, n, d)
    return LAMBDA_PARAM * inv + MU_PARAM * var + NU_PARAM * cov


def _vicreg_kernel(fa_ref, na_ref, g0_ref, g1_ref, out_ref, acc_ref):
    p = pl.program_id(0)

    @pl.when(p == 0)
    def _():
        acc_ref[0] = ALPHA * _vicreg_pair(g0_ref[...], g1_ref[...], 64.0, float(D), True)

    n = float(64 * K)
    acc_ref[0] += (1.0 - ALPHA) * 0.5 * _vicreg_pair(
        fa_ref[0], na_ref[0], n, float(C), False)

    @pl.when(p == 3)
    def _():
        out_ref[0] = acc_ref[0]


@jax.jit
def _vicregl(z_global0, z_global1, z_local0, z_local1, grid0, grid1):
    B = z_local0.shape[0]
    za = z_local0.reshape(B, L, C)
    zb = z_local1.reshape(B, L, C)
    ga = grid0.reshape(B, L, 2)
    gb = grid1.reshape(B, L, 2)

    fa, na = pl.pallas_call(
        _match_kernel,
        grid=(B,),
        in_specs=[
            pl.BlockSpec((1, L, C), lambda b: (b, 0, 0)),
            pl.BlockSpec((1, L, C), lambda b: (b, 0, 0)),
            pl.BlockSpec((1, L, 2), lambda b: (b, 0, 0)),
            pl.BlockSpec((1, L, 2), lambda b: (b, 0, 0)),
        ],
        out_specs=[
            pl.BlockSpec((4, 1, K, C), lambda b: (0, b, 0, 0)),
            pl.BlockSpec((4, 1, K, C), lambda b: (0, b, 0, 0)),
        ],
        out_shape=[
            jax.ShapeDtypeStruct((4, B, K, C), jnp.float32),
            jax.ShapeDtypeStruct((4, B, K, C), jnp.float32),
        ],
    )(za, zb, ga, gb)

    fa = fa.reshape(4, B * K, C)
    na = na.reshape(4, B * K, C)

    out = pl.pallas_call(
        _vicreg_kernel,
        grid=(4,),
        in_specs=[
            pl.BlockSpec((1, B * K, C), lambda p: (p, 0, 0)),
            pl.BlockSpec((1, B * K, C), lambda p: (p, 0, 0)),
            pl.BlockSpec((B, D), lambda p: (0, 0)),
            pl.BlockSpec((B, D), lambda p: (0, 0)),
        ],
        out_specs=pl.BlockSpec(memory_space=pltpu.SMEM),
        out_shape=jax.ShapeDtypeStruct((1,), jnp.float32),
        scratch_shapes=[pltpu.SMEM((1,), jnp.float32)],
    )(fa, na, z_global0, z_global1)

    return out[0]


def kernel(z_global0, z_global1, z_local0, z_local1, grid0, grid1):
    return _vicregl(z_global0, z_global1, z_local0, z_local1, grid0, grid1)
